# baseline (speedup 1.0000x reference)
import functools

import jax
import jax.numpy as jnp
from jax import lax
from jax.experimental import pallas as pl
from jax.experimental.pallas import tpu as pltpu
from jax.experimental.pallas import tpu_sc as plsc

N = 10000
NP = 10240
IN_F = 5
HID = 256
K = 122
NG = 128
NGP = 256
E = 320000
NCORE = 2
NSUB = 16
NW = NCORE * NSUB
CB = 128
NCH = 80
GRP = 4
NGRPS = NCH // GRP
TPW = NCH * CB
EPAD = TPW * NW
EROWS = EPAD // CB
NPT = NP // NSUB
C0SZ = NG * NP
C0PT = C0SZ // NSUB
HIGH = lax.Precision.HIGHEST


def _rsqrt3(x):
    i = plsc.bitcast(x, jnp.int32)
    y = plsc.bitcast(jnp.int32(0x5F3759DF) - lax.shift_right_arithmetic(i, 1),
                     jnp.float32)
    for _ in range(3):
        y = y * (1.5 - 0.5 * x * y * y)
    return y


def _hist_body(dst_ref, bhist_ref, btbl_ref, xpf_ref,
               dinv_out, ptab_out, wdst_out, y_out, occ_out,
               buf_v, hist_v, bbuf_v, chist_v, red_v, cred_v, invc_v,
               dinv_v, wdst_v, ptabs_v, bt_v, xp_v, y_v, occ_v,
               deg_sh, cnt_sh):
    sid = lax.axis_index("s")
    z16 = jnp.zeros((16,), jnp.float32)
    o16 = jnp.ones((16,), jnp.float32)
    iota = lax.iota(jnp.int32, 16)
    ipt = EPAD // NSUB

    def zero_hist(i, c):
        hist_v[pl.ds(i * 16, 16)] = z16
        return c
    lax.fori_loop(0, NP // 16, zero_hist, 0)

    def zero_chist(i, c):
        chist_v[pl.ds(i * 16, 16)] = z16
        return c
    lax.fori_loop(0, NGP // 16, zero_chist, 0)

    pltpu.sync_copy(dst_ref.at[pl.ds(sid * ipt, ipt)], buf_v)

    def scat(i, c):
        idx = buf_v[pl.ds(i * 16, 16)]
        plsc.addupdate_scatter(hist_v, [idx], o16)
        return c
    lax.fori_loop(0, ipt // 16, scat, 0)

    pltpu.sync_copy(bhist_ref.at[pl.ds(sid * NPT, NPT)], bbuf_v)

    def bscat(i, c):
        idx = bbuf_v[pl.ds(i * 16, 16)]
        plsc.addupdate_scatter(chist_v, [idx], o16)
        return c
    lax.fori_loop(0, NPT // 16, bscat, 0)

    pltpu.sync_copy(hist_v, deg_sh.at[sid])
    pltpu.sync_copy(chist_v, cnt_sh.at[sid])
    plsc.subcore_barrier()

    for r in range(NSUB):
        pltpu.sync_copy(deg_sh.at[r, pl.ds(sid * NPT, NPT)], red_v.at[r])
    pltpu.sync_copy(cnt_sh, cred_v)
    pltpu.sync_copy(btbl_ref.at[pl.ds(sid * NPT, NPT)], bt_v)
    pltpu.sync_copy(xpf_ref.at[pl.ds(sid * NPT * 8, NPT * 8)], xp_v)

    def credf(i, c):
        s = cred_v[0, pl.ds(i * 16, 16)]
        for r in range(1, NSUB):
            s = s + cred_v[r, pl.ds(i * 16, 16)]
        rc = _rsqrt3(jnp.maximum(s, 1.0))
        invc_v[pl.ds(i * 16, 16)] = rc * rc
        return c
    lax.fori_loop(0, NGP // 16, credf, 0)

    def norm(i, c):
        s = red_v[0, pl.ds(i * 16, 16)]
        for r in range(1, NSUB):
            s = s + red_v[r, pl.ds(i * 16, 16)]
        nid = sid * NPT + i * 16 + iota
        deg = s + jnp.where(nid < N, 1.0, 0.0)
        dv = jnp.where(deg > 0, _rsqrt3(deg), 0.0)
        dinv_v[pl.ds(i * 16, 16)] = dv
        bt16 = bt_v[pl.ds(i * 16, 16)]
        iv = plsc.load_gather(invc_v, [bt16])
        wv = jnp.where(nid < N, dv * iv, 0.0)
        wdst_v[pl.ds(i * 16, 16)] = wv
        ptabs_v[pl.ds(i * 16, 16)] = (
            ((plsc.bitcast(wv, jnp.int32) + 256) & jnp.int32(-512)) | bt16)
        return c
    lax.fori_loop(0, NPT // 16, norm, 0)

    def yrow(j, c):
        dexp = plsc.load_gather(dinv_v, [lax.shift_right_arithmetic(iota, 3)
                                         + 2 * j])
        y_v[pl.ds(j * 16, 16)] = xp_v[pl.ds(j * 16, 16)] * dexp
        return c
    lax.fori_loop(0, NPT * 8 // 16, yrow, 0)

    pltpu.sync_copy(dinv_v, dinv_out.at[pl.ds(sid * NPT, NPT)])
    pltpu.sync_copy(wdst_v, wdst_out.at[pl.ds(sid * NPT, NPT)])
    pltpu.sync_copy(ptabs_v, ptab_out.at[pl.ds(sid * NPT, NPT)])
    pltpu.sync_copy(y_v, y_out.at[pl.ds(sid * NPT * 8, NPT * 8)])

    @pl.when(sid == 0)
    def _():
        def occf(i, c):
            s = cred_v[0, pl.ds(i * 16, 16)]
            for r in range(1, NSUB):
                s = s + cred_v[r, pl.ds(i * 16, 16)]
            occ_v[pl.ds(i * 16, 16)] = jnp.where(s > 0, 1.0, 0.0)
            return c
        lax.fori_loop(0, NG // 16, occf, 0)
        pltpu.sync_copy(occ_v, occ_out)


def _edge_body(items_ref, y_ref, ptab_ref, zacc_ref, zc0_ref,
               acc_out, c0_out,
               ebuf_v, ptab_v, msg_v, wgth_v, flat_v,
               y_sh, acc_sh, c0_sh, sem_g, sem_s):
    core = lax.axis_index("c")
    sid = lax.axis_index("s")
    wid = sid * NCORE + core

    pltpu.sync_copy(zacc_ref, acc_sh.at[pl.ds(sid * NPT, NPT)])
    pltpu.sync_copy(zc0_ref, c0_sh.at[pl.ds(sid * C0PT, C0PT)])

    pltpu.sync_copy(y_ref.at[pl.ds(sid * NPT, NPT)],
                    y_sh.at[pl.ds(sid * NPT, NPT)])
    pltpu.sync_copy(ptab_ref, ptab_v)
    plsc.subcore_barrier()

    def fire_gathers(slot, grp):
        hs = []
        for b in range(GRP):
            hs.append(pltpu.async_copy(
                y_sh.at[ebuf_v.at[slot, b, 0]], msg_v.at[slot, b], sem_g))
        return hs

    def load_idx(slot, grp):
        pltpu.sync_copy(items_ref.at[pl.ds(wid * NCH + grp * GRP, GRP)],
                        ebuf_v.at[slot])

    load_idx(0, 0)
    fire_gathers(0, 0)

    def group(g, carry):
        slot = lax.rem(g, 2)
        slot2 = 1 - slot
        for b in range(GRP):
            pltpu.make_async_copy(
                y_sh.at[ebuf_v.at[slot, b, 0]], msg_v.at[slot, b],
                sem_g).wait()
        gnext = jnp.minimum(g + 1, NGRPS - 1)
        load_idx(slot2, gnext)
        fire_gathers(slot2, gnext)
        for b in range(GRP):
            for j in range(CB // 16):
                d16 = ebuf_v[slot, b, 1, pl.ds(j * 16, 16)]
                s16 = ebuf_v[slot, b, 0, pl.ds(j * 16, 16)]
                word = plsc.load_gather(ptab_v, [d16])
                flat_v[slot, b, pl.ds(j * 16, 16)] = (
                    (word & jnp.int32(511)) * NP + s16)
                wgth_v[slot, b, pl.ds(j * 16, 16)] = plsc.bitcast(
                    word & jnp.int32(-512), jnp.float32)
        sh = []
        for b in range(GRP):
            sh.append(pltpu.async_copy(
                msg_v.at[slot, b], acc_sh.at[ebuf_v.at[slot, b, 1]],
                sem_s, add=True))
            sh.append(pltpu.async_copy(
                wgth_v.at[slot, b], c0_sh.at[flat_v.at[slot, b]],
                sem_s, add=True))
        for h in sh:
            h.wait()
        return carry
    lax.fori_loop(0, NGRPS, group, 0)

    lastslot = NGRPS % 2
    for b in range(GRP):
        pltpu.make_async_copy(
            y_sh.at[ebuf_v.at[lastslot, b, 0]], msg_v.at[lastslot, b],
            sem_g).wait()

    plsc.subcore_barrier()
    pltpu.sync_copy(acc_sh.at[pl.ds(sid * NPT, NPT)],
                    acc_out.at[core, pl.ds(sid * NPT, NPT)])
    pltpu.sync_copy(c0_sh.at[pl.ds(sid * C0PT, C0PT)],
                    c0_out.at[core, pl.ds(sid * C0PT, C0PT)])


KB2 = 2048
NB2 = NP // KB2


def _dense_body(accp, dinv_ref, y_ref, wdst_ref, batch2, c0_ref, w1_ref,
                b1_ref, w2_ref, wl_ref, b2_ref, bl_ref, occ_ref,
                out_ref, m_ref):
    j = pl.program_id(0)

    @pl.when(j == 0)
    def _():
        m_ref[...] = jnp.zeros_like(m_ref)

    acc = accp[0] + accp[1] + y_ref[...]
    dinv = dinv_ref[...]
    h1 = jnp.dot(acc * dinv, w1_ref[...], precision=HIGH) + b1_ref[...]
    r = jnp.maximum(h1, 0.0)
    nid = j * KB2 + lax.broadcasted_iota(jnp.int32, (KB2, 1), 0)
    rd = jnp.where(nid < N, r * dinv, 0.0)
    c0b = c0_ref[0] + c0_ref[1]
    m_ref[...] += jnp.dot(c0b, rd, precision=HIGH)
    onehot = (batch2[...] == lax.broadcasted_iota(jnp.int32, (KB2, NG), 1))
    m_ref[...] += lax.dot_general(
        onehot.astype(jnp.float32), wdst_ref[...] * rd,
        (((0,), (0,)), ((), ())), precision=HIGH)

    @pl.when(j == NB2 - 1)
    def _():
        w2l = jnp.dot(w2_ref[...], wl_ref[...], precision=HIGH)
        bl2 = jnp.dot(b2_ref[...], wl_ref[...], precision=HIGH)
        out_ref[...] = (jnp.dot(m_ref[...], w2l, precision=HIGH)
                        + occ_ref[...] * bl2 + bl_ref[...])


def kernel(x, edge_index, batch, W1, b1, W2, b2, Wlin, blin):
    f32 = jnp.float32
    ei = edge_index.astype(jnp.int32)
    bt = batch.astype(jnp.int32)
    pad = N + jnp.arange(EPAD - E, dtype=jnp.int32) % (NP - N)
    src_flat = jnp.concatenate([ei[0], pad])
    dst_flat = jnp.concatenate([ei[1], pad])
    items = jnp.stack([src_flat.reshape(EROWS, CB),
                       dst_flat.reshape(EROWS, CB)], axis=1)
    bhist = jnp.concatenate([bt, jnp.full((NP - N,), NG, jnp.int32)])
    btbl = jnp.concatenate([bt, jnp.zeros((NP - N,), jnp.int32)])
    xp = jnp.zeros((NP, 8), f32).at[:N, :IN_F].set(x.astype(f32))
    w1p = jnp.zeros((8, HID), f32).at[:IN_F].set(W1.astype(f32))
    zacc = jnp.zeros((NPT, 8), f32)
    zc0 = jnp.zeros((C0PT,), f32)

    mesh = plsc.VectorSubcoreMesh(core_axis_name="c", subcore_axis_name="s")
    sc_params = pltpu.CompilerParams(needs_layout_passes=False,
                                     use_tc_tiling_on_sc=False)

    dinv_f, ptab_f, wdst_f, y_f, occ_f = pl.kernel(
        _hist_body,
        compiler_params=sc_params,
        out_type=[jax.ShapeDtypeStruct((NP,), f32),
                  jax.ShapeDtypeStruct((NP,), jnp.int32),
                  jax.ShapeDtypeStruct((NP,), f32),
                  jax.ShapeDtypeStruct((NP * 8,), f32),
                  jax.ShapeDtypeStruct((NG,), f32)],
        mesh=mesh,
        scratch_types=[
            pltpu.VMEM((EPAD // NSUB,), jnp.int32),
            pltpu.VMEM((NP,), f32),
            pltpu.VMEM((NPT,), jnp.int32),
            pltpu.VMEM((NGP,), f32),
            pltpu.VMEM((NSUB, NPT), f32),
            pltpu.VMEM((NSUB, NGP), f32),
            pltpu.VMEM((NGP,), f32),
            pltpu.VMEM((NPT,), f32),
            pltpu.VMEM((NPT,), f32),
            pltpu.VMEM((NPT,), jnp.int32),
            pltpu.VMEM((NPT,), jnp.int32),
            pltpu.VMEM((NPT * 8,), f32),
            pltpu.VMEM((NPT * 8,), f32),
            pltpu.VMEM((NG,), f32),
            pltpu.VMEM_SHARED((NSUB, NP), f32),
            pltpu.VMEM_SHARED((NSUB, NGP), f32),
        ],
    )(dst_flat, bhist, btbl, xp.reshape(NP * 8))
    yarr = y_f.reshape(NP, 8)

    acc_part, c0_part = pl.kernel(
        _edge_body,
        compiler_params=sc_params,
        out_type=[jax.ShapeDtypeStruct((NCORE, NP, 8), f32),
                  jax.ShapeDtypeStruct((NCORE, C0SZ), f32)],
        mesh=mesh,
        scratch_types=[
            pltpu.VMEM((2, GRP, 2, CB), jnp.int32),
            pltpu.VMEM((NP,), jnp.int32),
            pltpu.VMEM((2, GRP, CB, 8), f32),
            pltpu.VMEM((2, GRP, CB), f32),
            pltpu.VMEM((2, GRP, CB), jnp.int32),
            pltpu.VMEM_SHARED((NP, 8), f32),
            pltpu.VMEM_SHARED((NP, 8), f32),
            pltpu.VMEM_SHARED((C0SZ,), f32),
            pltpu.SemaphoreType.DMA,
            pltpu.SemaphoreType.DMA,
        ],
    )(items, yarr, ptab_f, zacc, zc0)

    out = pl.pallas_call(
        _dense_body,
        grid=(NB2,),
        in_specs=[
            pl.BlockSpec((NCORE, KB2, 8), lambda j: (0, j, 0)),
            pl.BlockSpec((KB2, 1), lambda j: (j, 0)),
            pl.BlockSpec((KB2, 8), lambda j: (j, 0)),
            pl.BlockSpec((KB2, 1), lambda j: (j, 0)),
            pl.BlockSpec((KB2, 1), lambda j: (j, 0)),
            pl.BlockSpec((NCORE, NG, KB2), lambda j: (0, 0, j)),
            pl.BlockSpec((8, HID), lambda j: (0, 0)),
            pl.BlockSpec((1, HID), lambda j: (0, 0)),
            pl.BlockSpec((HID, HID), lambda j: (0, 0)),
            pl.BlockSpec((HID, K), lambda j: (0, 0)),
            pl.BlockSpec((1, HID), lambda j: (0, 0)),
            pl.BlockSpec((1, K), lambda j: (0, 0)),
            pl.BlockSpec((NG, 1), lambda j: (0, 0)),
        ],
        out_specs=pl.BlockSpec((NG, K), lambda j: (0, 0)),
        out_shape=jax.ShapeDtypeStruct((NG, K), f32),
        scratch_shapes=[pltpu.VMEM((NG, HID), f32)],
    )(acc_part, dinv_f.reshape(NP, 1), yarr, wdst_f.reshape(NP, 1),
      btbl.reshape(NP, 1), c0_part.reshape(NCORE, NG, NP),
      w1p, b1.astype(f32).reshape(1, HID), W2.astype(f32),
      Wlin.astype(f32), b2.astype(f32).reshape(1, HID),
      blin.astype(f32).reshape(1, K), occ_f.reshape(NG, 1))
    return out


# speedup vs baseline: 1.0884x; 1.0884x over previous
import functools

import jax
import jax.numpy as jnp
from jax import lax
from jax.experimental import pallas as pl
from jax.experimental.pallas import tpu as pltpu
from jax.experimental.pallas import tpu_sc as plsc

N = 10000
NP = 10240
IN_F = 5
HID = 256
K = 122
NG = 128
NGP = 256
E = 320000
NCORE = 2
NSUB = 16
NW = NCORE * NSUB
CB = 128
NCH = 80
GRP = 4
NGRPS = NCH // GRP
TPW = NCH * CB
EPAD = TPW * NW
EROWS = EPAD // CB
NPT = NP // NSUB
C0SZ = NG * NP
C0PT = C0SZ // NSUB
HIGH = lax.Precision.HIGHEST


def _rsqrt3(x):
    i = plsc.bitcast(x, jnp.int32)
    y = plsc.bitcast(jnp.int32(0x5F3759DF) - lax.shift_right_arithmetic(i, 1),
                     jnp.float32)
    for _ in range(3):
        y = y * (1.5 - 0.5 * x * y * y)
    return y


def _hist_body(dst_ref, bhist_ref, btbl_ref, xpf_ref,
               dinv_out, ptab_out, wdst_out, y_out, occ_out,
               buf_v, hist_v, bbuf_v, chist_v, red_v, cred_v, invc_v,
               dinv_v, wdst_v, ptabs_v, bt_v, xp_v, y_v, occ_v,
               deg_sh, cnt_sh):
    sid = lax.axis_index("s")
    z16 = jnp.zeros((16,), jnp.float32)
    o16 = jnp.ones((16,), jnp.float32)
    iota = lax.iota(jnp.int32, 16)
    ipt = EPAD // NSUB

    def zero_hist(i, c):
        hist_v[pl.ds(i * 16, 16)] = z16
        return c
    lax.fori_loop(0, NP // 16, zero_hist, 0)

    def zero_chist(i, c):
        chist_v[pl.ds(i * 16, 16)] = z16
        return c
    lax.fori_loop(0, NGP // 16, zero_chist, 0)

    pltpu.sync_copy(dst_ref.at[pl.ds(sid * ipt, ipt)], buf_v)

    def scat(i, c):
        idx = buf_v[pl.ds(i * 16, 16)]
        plsc.addupdate_scatter(hist_v, [idx], o16)
        return c
    lax.fori_loop(0, ipt // 16, scat, 0)

    pltpu.sync_copy(bhist_ref.at[pl.ds(sid * NPT, NPT)], bbuf_v)

    def bscat(i, c):
        idx = bbuf_v[pl.ds(i * 16, 16)]
        plsc.addupdate_scatter(chist_v, [idx], o16)
        return c
    lax.fori_loop(0, NPT // 16, bscat, 0)

    pltpu.sync_copy(hist_v, deg_sh.at[sid])
    pltpu.sync_copy(chist_v, cnt_sh.at[sid])
    plsc.subcore_barrier()

    for r in range(NSUB):
        pltpu.sync_copy(deg_sh.at[r, pl.ds(sid * NPT, NPT)], red_v.at[r])
    pltpu.sync_copy(cnt_sh, cred_v)
    pltpu.sync_copy(btbl_ref.at[pl.ds(sid * NPT, NPT)], bt_v)
    pltpu.sync_copy(xpf_ref.at[pl.ds(sid * NPT * 8, NPT * 8)], xp_v)

    def credf(i, c):
        s = cred_v[0, pl.ds(i * 16, 16)]
        for r in range(1, NSUB):
            s = s + cred_v[r, pl.ds(i * 16, 16)]
        rc = _rsqrt3(jnp.maximum(s, 1.0))
        invc_v[pl.ds(i * 16, 16)] = rc * rc
        return c
    lax.fori_loop(0, NGP // 16, credf, 0)

    def norm(i, c):
        s = red_v[0, pl.ds(i * 16, 16)]
        for r in range(1, NSUB):
            s = s + red_v[r, pl.ds(i * 16, 16)]
        nid = sid * NPT + i * 16 + iota
        deg = s + jnp.where(nid < N, 1.0, 0.0)
        dv = jnp.where(deg > 0, _rsqrt3(deg), 0.0)
        dinv_v[pl.ds(i * 16, 16)] = dv
        bt16 = bt_v[pl.ds(i * 16, 16)]
        iv = plsc.load_gather(invc_v, [bt16])
        wv = jnp.where(nid < N, dv * iv, 0.0)
        wdst_v[pl.ds(i * 16, 16)] = wv
        ptabs_v[pl.ds(i * 16, 16)] = (
            ((plsc.bitcast(wv, jnp.int32) + 256) & jnp.int32(-512)) | bt16)
        return c
    lax.fori_loop(0, NPT // 16, norm, 0)

    def yrow(j, c):
        dexp = plsc.load_gather(dinv_v, [lax.shift_right_arithmetic(iota, 3)
                                         + 2 * j])
        y_v[pl.ds(j * 16, 16)] = xp_v[pl.ds(j * 16, 16)] * dexp
        return c
    lax.fori_loop(0, NPT * 8 // 16, yrow, 0)

    pltpu.sync_copy(dinv_v, dinv_out.at[pl.ds(sid * NPT, NPT)])
    pltpu.sync_copy(wdst_v, wdst_out.at[pl.ds(sid * NPT, NPT)])
    pltpu.sync_copy(ptabs_v, ptab_out.at[pl.ds(sid * NPT, NPT)])
    pltpu.sync_copy(y_v, y_out.at[pl.ds(sid * NPT * 8, NPT * 8)])

    @pl.when(sid == 0)
    def _():
        def occf(i, c):
            s = cred_v[0, pl.ds(i * 16, 16)]
            for r in range(1, NSUB):
                s = s + cred_v[r, pl.ds(i * 16, 16)]
            occ_v[pl.ds(i * 16, 16)] = jnp.where(s > 0, 1.0, 0.0)
            return c
        lax.fori_loop(0, NG // 16, occf, 0)
        pltpu.sync_copy(occ_v, occ_out)


def _edge_body(items_ref, y_ref, ptab_ref, zacc_ref, zc0_ref,
               acc_out, c0_out,
               ebuf_v, ptab_v, msg_v, wgth_v, flat_v,
               y_sh, acc_sh, c0_sh, sem_g, sem_s):
    core = lax.axis_index("c")
    sid = lax.axis_index("s")
    wid = sid * NCORE + core

    pltpu.sync_copy(zacc_ref, acc_sh.at[pl.ds(sid * NPT, NPT)])
    pltpu.sync_copy(zc0_ref, c0_sh.at[pl.ds(sid * C0PT, C0PT)])

    pltpu.sync_copy(y_ref.at[pl.ds(sid * NPT, NPT)],
                    y_sh.at[pl.ds(sid * NPT, NPT)])
    pltpu.sync_copy(ptab_ref, ptab_v)
    plsc.subcore_barrier()

    def fire_gathers(slot, grp):
        hs = []
        for b in range(GRP):
            hs.append(pltpu.async_copy(
                y_sh.at[ebuf_v.at[slot, b, 0]], msg_v.at[slot, b], sem_g))
        return hs

    def load_idx(slot, grp):
        pltpu.sync_copy(items_ref.at[pl.ds(wid * NCH + grp * GRP, GRP)],
                        ebuf_v.at[slot])

    load_idx(0, 0)
    fire_gathers(0, 0)

    def group(g, carry):
        slot = lax.rem(g, 2)
        slot2 = 1 - slot
        for b in range(GRP):
            pltpu.make_async_copy(
                y_sh.at[ebuf_v.at[slot, b, 0]], msg_v.at[slot, b],
                sem_g).wait()
        gnext = jnp.minimum(g + 1, NGRPS - 1)
        load_idx(slot2, gnext)
        fire_gathers(slot2, gnext)
        for b in range(GRP):
            for j in range(CB // 16):
                d16 = ebuf_v[slot, b, 1, pl.ds(j * 16, 16)]
                s16 = ebuf_v[slot, b, 0, pl.ds(j * 16, 16)]
                word = plsc.load_gather(ptab_v, [d16])
                flat_v[slot, b, pl.ds(j * 16, 16)] = (
                    (word & jnp.int32(511)) * NP + s16)
                wgth_v[slot, b, pl.ds(j * 16, 16)] = plsc.bitcast(
                    word & jnp.int32(-512), jnp.float32)
        sh = []
        for b in range(GRP):
            sh.append(pltpu.async_copy(
                msg_v.at[slot, b], acc_sh.at[ebuf_v.at[slot, b, 1]],
                sem_s, add=True))
            sh.append(pltpu.async_copy(
                wgth_v.at[slot, b], c0_sh.at[flat_v.at[slot, b]],
                sem_s, add=True))
        for h in sh:
            h.wait()
        return carry
    lax.fori_loop(0, NGRPS, group, 0)

    lastslot = NGRPS % 2
    for b in range(GRP):
        pltpu.make_async_copy(
            y_sh.at[ebuf_v.at[lastslot, b, 0]], msg_v.at[lastslot, b],
            sem_g).wait()

    @pl.when(core == 0)
    def _():
        iota = lax.iota(jnp.int32, 16)

        def sscat(q, c):
            base = sid * NPT + q * CB
            for j in range(CB // 16):
                n16 = base + j * 16 + iota
                word = ptab_v[pl.ds(base + j * 16, 16)]
                flat_v[0, 0, pl.ds(j * 16, 16)] = (
                    (word & jnp.int32(511)) * NP + n16)
                wgth_v[0, 0, pl.ds(j * 16, 16)] = plsc.bitcast(
                    word & jnp.int32(-512), jnp.float32)
            pltpu.async_copy(wgth_v.at[0, 0], c0_sh.at[flat_v.at[0, 0]],
                             sem_s, add=True).wait()
            return c
        lax.fori_loop(0, NPT // CB, sscat, 0)

    plsc.subcore_barrier()
    pltpu.sync_copy(acc_sh.at[pl.ds(sid * NPT, NPT)],
                    acc_out.at[core, pl.ds(sid * NPT, NPT)])
    pltpu.sync_copy(c0_sh.at[pl.ds(sid * C0PT, C0PT)],
                    c0_out.at[core, pl.ds(sid * C0PT, C0PT)])


KB2 = 2048
NB2 = NP // KB2


def _dense_body(accp, dinv_ref, y_ref, c0_ref, w1_ref,
                b1_ref, w2_ref, wl_ref, b2_ref, bl_ref, occ_ref,
                out_ref, m_ref):
    j = pl.program_id(0)

    @pl.when(j == 0)
    def _():
        m_ref[...] = jnp.zeros_like(m_ref)

    acc = accp[0] + accp[1] + y_ref[...]
    dinv = dinv_ref[...]
    h1 = jnp.dot(acc * dinv, w1_ref[...], precision=HIGH) + b1_ref[...]
    r = jnp.maximum(h1, 0.0)
    nid = j * KB2 + lax.broadcasted_iota(jnp.int32, (KB2, 1), 0)
    rd = jnp.where(nid < N, r * dinv, 0.0)
    c0b = c0_ref[0] + c0_ref[1]
    m_ref[...] += jnp.dot(c0b, rd, precision=HIGH)

    @pl.when(j == NB2 - 1)
    def _():
        w2l = jnp.dot(w2_ref[...], wl_ref[...], precision=HIGH)
        bl2 = jnp.dot(b2_ref[...], wl_ref[...], precision=HIGH)
        out_ref[...] = (jnp.dot(m_ref[...], w2l, precision=HIGH)
                        + occ_ref[...] * bl2 + bl_ref[...])


def kernel(x, edge_index, batch, W1, b1, W2, b2, Wlin, blin):
    f32 = jnp.float32
    ei = edge_index.astype(jnp.int32)
    bt = batch.astype(jnp.int32)
    pad = N + jnp.arange(EPAD - E, dtype=jnp.int32) % (NP - N)
    src_flat = jnp.concatenate([ei[0], pad])
    dst_flat = jnp.concatenate([ei[1], pad])
    items = jnp.stack([src_flat.reshape(EROWS, CB),
                       dst_flat.reshape(EROWS, CB)], axis=1)
    bhist = jnp.concatenate([bt, jnp.full((NP - N,), NG, jnp.int32)])
    btbl = jnp.concatenate([bt, jnp.zeros((NP - N,), jnp.int32)])
    xp = jnp.zeros((NP, 8), f32).at[:N, :IN_F].set(x.astype(f32))
    w1p = jnp.zeros((8, HID), f32).at[:IN_F].set(W1.astype(f32))
    zacc = jnp.zeros((NPT, 8), f32)
    zc0 = jnp.zeros((C0PT,), f32)

    mesh = plsc.VectorSubcoreMesh(core_axis_name="c", subcore_axis_name="s")
    sc_params = pltpu.CompilerParams(needs_layout_passes=False,
                                     use_tc_tiling_on_sc=False)

    dinv_f, ptab_f, wdst_f, y_f, occ_f = pl.kernel(
        _hist_body,
        compiler_params=sc_params,
        out_type=[jax.ShapeDtypeStruct((NP,), f32),
                  jax.ShapeDtypeStruct((NP,), jnp.int32),
                  jax.ShapeDtypeStruct((NP,), f32),
                  jax.ShapeDtypeStruct((NP * 8,), f32),
                  jax.ShapeDtypeStruct((NG,), f32)],
        mesh=mesh,
        scratch_types=[
            pltpu.VMEM((EPAD // NSUB,), jnp.int32),
            pltpu.VMEM((NP,), f32),
            pltpu.VMEM((NPT,), jnp.int32),
            pltpu.VMEM((NGP,), f32),
            pltpu.VMEM((NSUB, NPT), f32),
            pltpu.VMEM((NSUB, NGP), f32),
            pltpu.VMEM((NGP,), f32),
            pltpu.VMEM((NPT,), f32),
            pltpu.VMEM((NPT,), f32),
            pltpu.VMEM((NPT,), jnp.int32),
            pltpu.VMEM((NPT,), jnp.int32),
            pltpu.VMEM((NPT * 8,), f32),
            pltpu.VMEM((NPT * 8,), f32),
            pltpu.VMEM((NG,), f32),
            pltpu.VMEM_SHARED((NSUB, NP), f32),
            pltpu.VMEM_SHARED((NSUB, NGP), f32),
        ],
    )(dst_flat, bhist, btbl, xp.reshape(NP * 8))
    yarr = y_f.reshape(NP, 8)

    acc_part, c0_part = pl.kernel(
        _edge_body,
        compiler_params=sc_params,
        out_type=[jax.ShapeDtypeStruct((NCORE, NP, 8), f32),
                  jax.ShapeDtypeStruct((NCORE, C0SZ), f32)],
        mesh=mesh,
        scratch_types=[
            pltpu.VMEM((2, GRP, 2, CB), jnp.int32),
            pltpu.VMEM((NP,), jnp.int32),
            pltpu.VMEM((2, GRP, CB, 8), f32),
            pltpu.VMEM((2, GRP, CB), f32),
            pltpu.VMEM((2, GRP, CB), jnp.int32),
            pltpu.VMEM_SHARED((NP, 8), f32),
            pltpu.VMEM_SHARED((NP, 8), f32),
            pltpu.VMEM_SHARED((C0SZ,), f32),
            pltpu.SemaphoreType.DMA,
            pltpu.SemaphoreType.DMA,
        ],
    )(items, yarr, ptab_f, zacc, zc0)

    out = pl.pallas_call(
        _dense_body,
        grid=(NB2,),
        in_specs=[
            pl.BlockSpec((NCORE, KB2, 8), lambda j: (0, j, 0)),
            pl.BlockSpec((KB2, 1), lambda j: (j, 0)),
            pl.BlockSpec((KB2, 8), lambda j: (j, 0)),
            pl.BlockSpec((NCORE, NG, KB2), lambda j: (0, 0, j)),
            pl.BlockSpec((8, HID), lambda j: (0, 0)),
            pl.BlockSpec((1, HID), lambda j: (0, 0)),
            pl.BlockSpec((HID, HID), lambda j: (0, 0)),
            pl.BlockSpec((HID, K), lambda j: (0, 0)),
            pl.BlockSpec((1, HID), lambda j: (0, 0)),
            pl.BlockSpec((1, K), lambda j: (0, 0)),
            pl.BlockSpec((NG, 1), lambda j: (0, 0)),
        ],
        out_specs=pl.BlockSpec((NG, K), lambda j: (0, 0)),
        out_shape=jax.ShapeDtypeStruct((NG, K), f32),
        scratch_shapes=[pltpu.VMEM((NG, HID), f32)],
    )(acc_part, dinv_f.reshape(NP, 1), yarr,
      c0_part.reshape(NCORE, NG, NP),
      w1p, b1.astype(f32).reshape(1, HID), W2.astype(f32),
      Wlin.astype(f32), b2.astype(f32).reshape(1, HID),
      blin.astype(f32).reshape(1, K), occ_f.reshape(NG, 1))
    return out
